# baseline (device time: 51832 ns/iter reference)
import numpy as np
import jax
import jax.numpy as jnp
from jax import lax
from jax.experimental import pallas as pl
from jax.experimental.pallas import tpu as pltpu

N_DEV = 4
B = 2
S_LOC = 128
S = S_LOC * N_DEV
D = 512
H_LOC = 4
DH = 64
HD = H_LOC * DH


def _rope_consts():
    inv = 1.0 / (10000.0 ** (np.arange(0, DH, 2) / DH))
    pos = np.arange(S)[:, None] * inv[None, :]
    cos = np.repeat(np.cos(pos), 2, axis=-1)
    sin = np.repeat(np.sin(pos), 2, axis=-1)
    cos = np.tile(cos, (1, H_LOC)).astype(np.float32)
    sin = np.tile(sin, (1, H_LOC)).astype(np.float32)
    P = np.zeros((HD, HD), np.float32)
    ev = np.arange(0, HD, 2)
    P[ev + 1, ev] = -1.0
    P[ev, ev + 1] = 1.0
    return cos, sin, P.astype(np.float32)


_COS, _SIN, _P = _rope_consts()


def kernel(x, Wq, Wk, Wv, Wo):
    def body(
        x_ref, wq_ref, wk_ref, wv_ref, wo_ref, cos_ref, sin_ref, p_ref,
        out_ref,
        xfull_ref, partial_ref, ag_comm, rs_comm,
        ag_send_sems, ag_recv_sems, rs_send_sems, rs_recv_sems,
    ):
        me = lax.axis_index("i")
        left = lax.rem(me + N_DEV - 1, N_DEV)
        right = lax.rem(me + 1, N_DEV)

        barrier_sem = pltpu.get_barrier_semaphore()
        for nbr in (left, right):
            pl.semaphore_signal(
                barrier_sem, inc=1,
                device_id=(nbr,), device_id_type=pl.DeviceIdType.MESH,
            )
        pl.semaphore_wait(barrier_sem, 2)

        xfull_ref[:, pl.ds(me * S_LOC, S_LOC), :] = x_ref[:, :, :]
        ag_comm[0, :, :, :] = x_ref[:, :, :]
        for h in range(N_DEV - 1):
            s_slot = h % 2
            r_slot = (h + 1) % 2
            rdma = pltpu.make_async_remote_copy(
                src_ref=ag_comm.at[s_slot],
                dst_ref=ag_comm.at[r_slot],
                send_sem=ag_send_sems.at[s_slot],
                recv_sem=ag_recv_sems.at[r_slot],
                device_id=(right,),
                device_id_type=pl.DeviceIdType.MESH,
            )
            rdma.start()
            rdma.wait()
            origin = lax.rem(me + N_DEV - (h + 1), N_DEV)
            xfull_ref[:, pl.ds(origin * S_LOC, S_LOC), :] = ag_comm[r_slot, :, :, :]

        cos = cos_ref[:, :]
        sin = sin_ref[:, :]
        for b in range(B):
            xb = xfull_ref[b, :, :]
            q = jnp.dot(xb, wq_ref[:, :], preferred_element_type=jnp.float32)
            k = jnp.dot(xb, wk_ref[:, :], preferred_element_type=jnp.float32)
            v = jnp.dot(xb, wv_ref[:, :], preferred_element_type=jnp.float32)
            q_rot = jnp.dot(
                q.astype(jnp.bfloat16), p_ref[:, :],
                preferred_element_type=jnp.float32,
            )
            k_rot = jnp.dot(
                k.astype(jnp.bfloat16), p_ref[:, :],
                preferred_element_type=jnp.float32,
            )
            qb = (q * cos + q_rot * sin).astype(jnp.bfloat16)
            kb = (k * cos + k_rot * sin).astype(jnp.bfloat16)
            vb = v.astype(jnp.bfloat16)
            acc = jnp.zeros((S, D), jnp.float32)
            for h in range(H_LOC):
                qh = qb[:, h * DH:(h + 1) * DH]
                kh = kb[:, h * DH:(h + 1) * DH]
                s = lax.dot_general(
                    qh, kh, (((1,), (1,)), ((), ())),
                    preferred_element_type=jnp.float32,
                ) * 0.125
                m = jnp.max(s, axis=-1, keepdims=True)
                e = jnp.exp(s - m)
                w = e / jnp.sum(e, axis=-1, keepdims=True)
                ctx = jnp.dot(
                    w.astype(jnp.bfloat16), vb[:, h * DH:(h + 1) * DH],
                    preferred_element_type=jnp.float32,
                )
                acc = acc + jnp.dot(
                    ctx.astype(jnp.bfloat16), wo_ref[h * DH:(h + 1) * DH, :],
                    preferred_element_type=jnp.float32,
                )
            partial_ref[b, :, :] = acc

        b0 = lax.rem(me + N_DEV - 1, N_DEV)
        rs_comm[0, :, :, :] = partial_ref[:, pl.ds(b0 * S_LOC, S_LOC), :]
        for t in range(N_DEV - 1):
            s_slot = t % 2
            r_slot = (t + 1) % 2
            if t > 0:
                d = lax.rem(me + 2 * N_DEV - 1 - t, N_DEV)
                rs_comm[s_slot, :, :, :] = (
                    rs_comm[s_slot, :, :, :]
                    + partial_ref[:, pl.ds(d * S_LOC, S_LOC), :]
                )
            rdma = pltpu.make_async_remote_copy(
                src_ref=rs_comm.at[s_slot],
                dst_ref=rs_comm.at[r_slot],
                send_sem=rs_send_sems.at[s_slot],
                recv_sem=rs_recv_sems.at[r_slot],
                device_id=(right,),
                device_id_type=pl.DeviceIdType.MESH,
            )
            rdma.start()
            rdma.wait()
        out_ref[:, :, :] = (
            rs_comm[(N_DEV - 1) % 2, :, :, :]
            + partial_ref[:, pl.ds(me * S_LOC, S_LOC), :]
        )

    return pl.pallas_call(
        body,
        out_shape=jax.ShapeDtypeStruct((B, S_LOC, D), jnp.float32),
        in_specs=[pl.BlockSpec(memory_space=pltpu.VMEM)] * 8,
        out_specs=pl.BlockSpec(memory_space=pltpu.VMEM),
        scratch_shapes=[
            pltpu.VMEM((B, S, D), jnp.bfloat16),
            pltpu.VMEM((B, S, D), jnp.float32),
            pltpu.VMEM((2, B, S_LOC, D), jnp.bfloat16),
            pltpu.VMEM((2, B, S_LOC, D), jnp.float32),
            pltpu.SemaphoreType.DMA((2,)),
            pltpu.SemaphoreType.DMA((2,)),
            pltpu.SemaphoreType.DMA((2,)),
            pltpu.SemaphoreType.DMA((2,)),
        ],
        compiler_params=pltpu.CompilerParams(collective_id=0),
    )(
        x.astype(jnp.bfloat16),
        Wq.astype(jnp.bfloat16),
        Wk.astype(jnp.bfloat16),
        Wv.astype(jnp.bfloat16),
        Wo.astype(jnp.bfloat16),
        _COS,
        _SIN,
        _P.astype(jnp.bfloat16),
    )


# device time: 40185 ns/iter; 1.2898x vs baseline; 1.2898x over previous
import numpy as np
import jax
import jax.numpy as jnp
from jax import lax
from jax.experimental import pallas as pl
from jax.experimental.pallas import tpu as pltpu

N_DEV = 4
B = 2
S_LOC = 128
S = S_LOC * N_DEV
D = 512
H_LOC = 4
DH = 64
HD = H_LOC * DH


def _rope_consts():
    inv = 1.0 / (10000.0 ** (np.arange(0, DH, 2) / DH))
    pos = np.arange(S)[:, None] * inv[None, :]
    cos = np.repeat(np.cos(pos), 2, axis=-1)
    sin = np.repeat(np.sin(pos), 2, axis=-1)
    cos = np.tile(cos, (1, H_LOC)).astype(np.float32)
    sin = np.tile(sin, (1, H_LOC)).astype(np.float32)
    P = np.zeros((HD, HD), np.float32)
    ev = np.arange(0, HD, 2)
    P[ev + 1, ev] = -1.0
    P[ev, ev + 1] = 1.0
    return cos, sin, P.astype(np.float32)


_COS, _SIN, _P = _rope_consts()


def kernel(x, Wq, Wk, Wv, Wo):
    def body(
        x_ref, wq_ref, wk_ref, wv_ref, wo_ref, cos_ref, sin_ref, p_ref,
        out_ref,
        ag_buf, q_all, k_all, v_all, pblk, rs_buf,
        ag_send, ag_recv, rs_send, rs_recv,
    ):
        me = lax.axis_index("i")
        left = lax.rem(me + N_DEV - 1, N_DEV)
        right = lax.rem(me + 1, N_DEV)

        barrier_sem = pltpu.get_barrier_semaphore()
        for nbr in (left, right):
            pl.semaphore_signal(
                barrier_sem, inc=1,
                device_id=(nbr,), device_id_type=pl.DeviceIdType.MESH,
            )
        pl.semaphore_wait(barrier_sem, 2)

        def qkv_block(src, origin):
            rows = pl.ds(origin * S_LOC, S_LOC)
            cos = cos_ref[rows, :]
            sin = sin_ref[rows, :]
            for b in range(B):
                xb = src[b]
                q = jnp.dot(xb, wq_ref[:, :], preferred_element_type=jnp.float32)
                k = jnp.dot(xb, wk_ref[:, :], preferred_element_type=jnp.float32)
                v = jnp.dot(xb, wv_ref[:, :], preferred_element_type=jnp.float32)
                q_rot = jnp.dot(q.astype(jnp.bfloat16), p_ref[:, :],
                                preferred_element_type=jnp.float32)
                k_rot = jnp.dot(k.astype(jnp.bfloat16), p_ref[:, :],
                                preferred_element_type=jnp.float32)
                q_all[b, rows, :] = (q * cos + q_rot * sin).astype(jnp.bfloat16)
                k_all[b, rows, :] = (k * cos + k_rot * sin).astype(jnp.bfloat16)
                v_all[b, rows, :] = v.astype(jnp.bfloat16)

        ag_buf[0] = x_ref[:, :, :]
        hops = []
        rdma = pltpu.make_async_remote_copy(
            src_ref=ag_buf.at[0], dst_ref=ag_buf.at[1],
            send_sem=ag_send.at[0], recv_sem=ag_recv.at[0],
            device_id=(right,), device_id_type=pl.DeviceIdType.MESH,
        )
        rdma.start()
        hops.append(rdma)
        qkv_block(ag_buf.at[0], me)
        for h in range(N_DEV - 1):
            hops[h].wait()
            if h < N_DEV - 2:
                rdma = pltpu.make_async_remote_copy(
                    src_ref=ag_buf.at[h + 1], dst_ref=ag_buf.at[h + 2],
                    send_sem=ag_send.at[h + 1], recv_sem=ag_recv.at[h + 1],
                    device_id=(right,), device_id_type=pl.DeviceIdType.MESH,
                )
                rdma.start()
                hops.append(rdma)
            origin = lax.rem(me + N_DEV - (h + 1), N_DEV)
            qkv_block(ag_buf.at[h + 1], origin)

        def attn_block(d):
            rows = pl.ds(d * S_LOC, S_LOC)
            for b in range(B):
                qd = q_all[b, rows, :]
                acc = jnp.zeros((S_LOC, D), jnp.float32)
                for h in range(H_LOC):
                    qh = qd[:, h * DH:(h + 1) * DH]
                    kh = k_all[b, :, h * DH:(h + 1) * DH]
                    s = lax.dot_general(
                        qh, kh, (((1,), (1,)), ((), ())),
                        preferred_element_type=jnp.float32,
                    ) * 0.125
                    m = jnp.max(s, axis=-1, keepdims=True)
                    e = jnp.exp(s - m)
                    w = e / jnp.sum(e, axis=-1, keepdims=True)
                    ctx = jnp.dot(
                        w.astype(jnp.bfloat16), v_all[b, :, h * DH:(h + 1) * DH],
                        preferred_element_type=jnp.float32,
                    )
                    acc = acc + jnp.dot(
                        ctx.astype(jnp.bfloat16), wo_ref[h * DH:(h + 1) * DH, :],
                        preferred_element_type=jnp.float32,
                    )
                pblk[b] = acc

        attn_block(lax.rem(me + N_DEV - 1, N_DEV))
        for b in range(B):
            rs_buf[0, b] = pblk[b].astype(jnp.bfloat16)
        rs_hop = pltpu.make_async_remote_copy(
            src_ref=rs_buf.at[0], dst_ref=rs_buf.at[1],
            send_sem=rs_send.at[0], recv_sem=rs_recv.at[0],
            device_id=(right,), device_id_type=pl.DeviceIdType.MESH,
        )
        rs_hop.start()
        for t in range(1, N_DEV - 1):
            attn_block(lax.rem(me + 2 * N_DEV - 1 - t, N_DEV))
            rs_hop.wait()
            for b in range(B):
                rs_buf[t, b] = (
                    rs_buf[t, b].astype(jnp.float32) + pblk[b]
                ).astype(jnp.bfloat16)
            rs_hop = pltpu.make_async_remote_copy(
                src_ref=rs_buf.at[t], dst_ref=rs_buf.at[t + 1],
                send_sem=rs_send.at[t], recv_sem=rs_recv.at[t],
                device_id=(right,), device_id_type=pl.DeviceIdType.MESH,
            )
            rs_hop.start()
        attn_block(me)
        rs_hop.wait()
        for b in range(B):
            out_ref[b] = rs_buf[N_DEV - 1, b].astype(jnp.float32) + pblk[b]

    return pl.pallas_call(
        body,
        out_shape=jax.ShapeDtypeStruct((B, S_LOC, D), jnp.float32),
        in_specs=[pl.BlockSpec(memory_space=pltpu.VMEM)] * 8,
        out_specs=pl.BlockSpec(memory_space=pltpu.VMEM),
        scratch_shapes=[
            pltpu.VMEM((N_DEV, B, S_LOC, D), jnp.bfloat16),
            pltpu.VMEM((B, S, HD), jnp.bfloat16),
            pltpu.VMEM((B, S, HD), jnp.bfloat16),
            pltpu.VMEM((B, S, HD), jnp.bfloat16),
            pltpu.VMEM((B, S_LOC, D), jnp.float32),
            pltpu.VMEM((N_DEV, B, S_LOC, D), jnp.bfloat16),
            pltpu.SemaphoreType.DMA((N_DEV - 1,)),
            pltpu.SemaphoreType.DMA((N_DEV - 1,)),
            pltpu.SemaphoreType.DMA((N_DEV - 1,)),
            pltpu.SemaphoreType.DMA((N_DEV - 1,)),
        ],
        compiler_params=pltpu.CompilerParams(collective_id=0),
    )(
        x.astype(jnp.bfloat16),
        Wq.astype(jnp.bfloat16),
        Wk.astype(jnp.bfloat16),
        Wv.astype(jnp.bfloat16),
        Wo.astype(jnp.bfloat16),
        _COS,
        _SIN,
        _P.astype(jnp.bfloat16),
    )
